# Initial kernel scaffold; baseline (speedup 1.0000x reference)
import jax
import jax.numpy as jnp
from jax.experimental import pallas as pl


def kernel(atomic_charges, distances, idx_i, idx_j):
    raise NotImplementedError("write your pallas kernel here")


# speedup vs baseline: 249.2273x; 249.2273x over previous
import functools

import jax
import jax.numpy as jnp
from jax import lax
from jax.experimental import pallas as pl
from jax.experimental.pallas import tpu as pltpu
from jax.experimental.pallas import tpu_sc as plsc

_N_NODES = 100000
_N_EDGES = 6400000
_SHORT_CUTOFF = 4.0
_LONG_CUTOFF = 12.0
_KEHALF = 7.199822675975274
_ILR2 = 1.0 / (_LONG_CUTOFF * _LONG_CUTOFF)
_TLC = 2.0 / _LONG_CUTOFF

_L = 16
_NC, _NS = 2, 16
_NW = _NC * _NS
_ROWS = _N_EDGES // 128
_R = 16
_CHUNKS = _ROWS // _R
_CH_PER_TILE = -(-_CHUNKS // _NW)
_ACC_N = 100352
_SLICE = _ACC_N // _NS
_ZCH = 1568


@functools.partial(
    pl.kernel,
    out_type=jax.ShapeDtypeStruct((_NC, _ACC_N), jnp.float32),
    mesh=plsc.VectorSubcoreMesh(core_axis_name="c", subcore_axis_name="s"),
    compiler_params=pltpu.CompilerParams(needs_layout_passes=False),
    scratch_types=[
        pltpu.VMEM((_N_NODES,), jnp.float32),
        pltpu.VMEM((_R, 128), jnp.float32),
        pltpu.VMEM((_R, 128), jnp.int32),
        pltpu.VMEM((_R, 128), jnp.int32),
        pltpu.VMEM((_R, 128), jnp.float32),
        pltpu.VMEM((_ZCH,), jnp.float32),
        pltpu.VMEM_SHARED((_ACC_N,), jnp.float32),
    ],
)
def _sc_electro(q_hbm, d_hbm, ii_hbm, ij_hbm, out_hbm,
                table, db, iib, ijb, eb, zb, acc):
    c = lax.axis_index("c")
    s = lax.axis_index("s")
    wid = s * _NC + c

    zeros = jnp.zeros((_L,), jnp.float32)

    @pl.loop(0, _ZCH // _L)
    def _zb(i):
        zb[pl.ds(i * _L, _L)] = zeros

    for t in range(_SLICE // _ZCH):
        pltpu.sync_copy(zb, acc.at[pl.ds(s * _SLICE + t * _ZCH, _ZCH)])

    pltpu.sync_copy(q_hbm, table)
    plsc.subcore_barrier()

    @pl.loop(0, _CH_PER_TILE)
    def _chunk(k):
        cidx = wid + k * _NW

        @pl.when(cidx < _CHUNKS)
        def _():
            row0 = cidx * _R
            pltpu.sync_copy(d_hbm.at[pl.ds(row0, _R)], db)
            pltpu.sync_copy(ii_hbm.at[pl.ds(row0, _R)], iib)
            pltpu.sync_copy(ij_hbm.at[pl.ds(row0, _R)], ijb)

            @pl.loop(0, _R)
            def _row(r):
                for col in range(128 // _L):
                    sl = pl.ds(col * _L, _L)
                    dv = db[r, sl]
                    qi = plsc.load_gather(table, [iib[r, sl]])
                    qj = plsc.load_gather(table, [ijb[r, sl]])
                    d2 = dv * dv + 1.0
                    y = plsc.bitcast(
                        jnp.int32(0x5F3759DF) - (plsc.bitcast(d2, jnp.int32) >> 1),
                        jnp.float32)
                    y = y * (1.5 - 0.5 * d2 * y * y)
                    y = y * (1.5 - 0.5 * d2 * y * y)
                    y = y * (1.5 - 0.5 * d2 * y * y)
                    e_ord = 1.0 / dv + dv * _ILR2 - _TLC
                    e_sh = y + d2 * y * _ILR2 - _TLC
                    x = dv * (1.0 / _SHORT_CUTOFF)
                    x2 = x * x
                    sw = 1.0 - (x2 * x) * (10.0 - 15.0 * x + 6.0 * x2)
                    sw = jnp.where(dv < _SHORT_CUTOFF, sw, 0.0)
                    e = (_KEHALF * qi) * qj * (e_sh + sw * (e_ord - e_sh))
                    e = jnp.where(dv <= _LONG_CUTOFF, e, 0.0)
                    eb[r, sl] = e
                pltpu.sync_copy(eb.at[r], acc.at[iib.at[r]], add=True)

    plsc.subcore_barrier()
    pltpu.sync_copy(acc.at[pl.ds(s * _SLICE, _SLICE)],
                    out_hbm.at[c, pl.ds(s * _SLICE, _SLICE)])


def kernel(atomic_charges, distances, idx_i, idx_j):
    d2d = distances.reshape(_ROWS, 128)
    ii2d = idx_i.reshape(_ROWS, 128)
    ij2d = idx_j.reshape(_ROWS, 128)
    partials = _sc_electro(atomic_charges, d2d, ii2d, ij2d)
    return partials[0, :_N_NODES] + partials[1, :_N_NODES]


# speedup vs baseline: 502.6838x; 2.0170x over previous
import functools

import jax
import jax.numpy as jnp
from jax import lax
from jax.experimental import pallas as pl
from jax.experimental.pallas import tpu as pltpu
from jax.experimental.pallas import tpu_sc as plsc

_N_NODES = 100000
_N_EDGES = 6400000
_SHORT_CUTOFF = 4.0
_LONG_CUTOFF = 12.0
_KEHALF = 7.199822675975274
_ILR2 = 1.0 / (_LONG_CUTOFF * _LONG_CUTOFF)
_TLC = 2.0 / _LONG_CUTOFF

_L = 16
_NC, _NS = 2, 16
_NW = _NC * _NS
_ROWS = _N_EDGES // 128
_R = 16
_CHUNKS = _ROWS // _R
_CH_PER_TILE = -(-_CHUNKS // _NW)
_ACC_N = 100352
_SLICE = _ACC_N // _NS
_ZCH = 1568


@functools.partial(
    pl.kernel,
    out_type=jax.ShapeDtypeStruct((_NC, _ACC_N), jnp.float32),
    mesh=plsc.VectorSubcoreMesh(core_axis_name="c", subcore_axis_name="s"),
    compiler_params=pltpu.CompilerParams(needs_layout_passes=False),
    scratch_types=[
        pltpu.VMEM((_N_NODES,), jnp.float32),
        pltpu.VMEM((_R, 128), jnp.float32),
        pltpu.VMEM((_R, 128), jnp.float32),
        pltpu.VMEM((_R, 128), jnp.int32),
        pltpu.VMEM((_R, 128), jnp.int32),
        pltpu.VMEM((_R, 128), jnp.int32),
        pltpu.VMEM((_R, 128), jnp.int32),
        pltpu.VMEM((_R, 128), jnp.float32),
        pltpu.VMEM((_R, 128), jnp.float32),
        pltpu.VMEM((_ZCH,), jnp.float32),
        pltpu.VMEM_SHARED((_ACC_N,), jnp.float32),
        pltpu.SemaphoreType.DMA,
        pltpu.SemaphoreType.DMA,
        pltpu.SemaphoreType.DMA,
    ],
)
def _sc_electro(q_hbm, d_hbm, ii_hbm, ij_hbm, out_hbm,
                table, db0, db1, iib0, iib1, ijb0, ijb1, eb0, eb1, zb, acc,
                insem0, insem1, scsem):
    c = lax.axis_index("c")
    s = lax.axis_index("s")
    wid = s * _NC + c
    bufs = ((db0, iib0, ijb0, eb0, insem0), (db1, iib1, ijb1, eb1, insem1))

    zeros = jnp.zeros((_L,), jnp.float32)

    @pl.loop(0, _ZCH // _L)
    def _zb(i):
        zb[pl.ds(i * _L, _L)] = zeros

    for t in range(_SLICE // _ZCH):
        pltpu.sync_copy(zb, acc.at[pl.ds(s * _SLICE + t * _ZCH, _ZCH)])

    pltpu.sync_copy(q_hbm, table)
    plsc.subcore_barrier()

    def start_fetch(cidx, buf):
        db, iib, ijb, _, insem = buf
        row0 = cidx * _R
        pltpu.async_copy(d_hbm.at[pl.ds(row0, _R)], db, insem)
        pltpu.async_copy(ii_hbm.at[pl.ds(row0, _R)], iib, insem)
        pltpu.async_copy(ij_hbm.at[pl.ds(row0, _R)], ijb, insem)

    def wait_fetch(cidx, buf):
        db, iib, ijb, _, insem = buf
        row0 = cidx * _R
        pltpu.make_async_copy(d_hbm.at[pl.ds(row0, _R)], db, insem).wait()
        pltpu.make_async_copy(ii_hbm.at[pl.ds(row0, _R)], iib, insem).wait()
        pltpu.make_async_copy(ij_hbm.at[pl.ds(row0, _R)], ijb, insem).wait()

    def compute_chunk(buf):
        db, iib, ijb, eb, _ = buf

        @pl.loop(0, _R)
        def _row(r):
            for col in range(128 // _L):
                sl = pl.ds(col * _L, _L)
                dv = db[r, sl]
                qi = plsc.load_gather(table, [iib[r, sl]])
                qj = plsc.load_gather(table, [ijb[r, sl]])
                d2 = dv * dv + 1.0
                y = plsc.bitcast(
                    jnp.int32(0x5F3759DF) - (plsc.bitcast(d2, jnp.int32) >> 1),
                    jnp.float32)
                y = y * (1.5 - 0.5 * d2 * y * y)
                y = y * (1.5 - 0.5 * d2 * y * y)
                e_ord = 1.0 / dv + dv * _ILR2 - _TLC
                e_sh = y * (1.0 + d2 * _ILR2) - _TLC
                x = dv * (1.0 / _SHORT_CUTOFF)
                x2 = x * x
                sw = 1.0 - (x2 * x) * (10.0 - 15.0 * x + 6.0 * x2)
                sw = jnp.where(dv < _SHORT_CUTOFF, sw, 0.0)
                e = (_KEHALF * qi) * qj * (e_sh + sw * (e_ord - e_sh))
                e = jnp.where(dv <= _LONG_CUTOFF, e, 0.0)
                eb[r, sl] = e
            pltpu.async_copy(eb.at[r], acc.at[iib.at[r]], scsem, add=True)

        @pl.loop(0, _R)
        def _drain(r):
            pltpu.make_async_copy(eb.at[r], acc.at[iib.at[r]], scsem).wait()

    start_fetch(wid, bufs[0])

    @pl.loop(0, _CH_PER_TILE, step=2)
    def _pair(k):
        c0 = wid + k * _NW
        c1 = c0 + _NW
        c2 = c1 + _NW

        @pl.when(c1 < _CHUNKS)
        def _():
            start_fetch(c1, bufs[1])

        wait_fetch(c0, bufs[0])
        compute_chunk(bufs[0])

        @pl.when(c2 < _CHUNKS)
        def _():
            start_fetch(c2, bufs[0])

        @pl.when(c1 < _CHUNKS)
        def _():
            wait_fetch(c1, bufs[1])
            compute_chunk(bufs[1])

    plsc.subcore_barrier()
    pltpu.sync_copy(acc.at[pl.ds(s * _SLICE, _SLICE)],
                    out_hbm.at[c, pl.ds(s * _SLICE, _SLICE)])


def kernel(atomic_charges, distances, idx_i, idx_j):
    d2d = distances.reshape(_ROWS, 128)
    ii2d = idx_i.reshape(_ROWS, 128)
    ij2d = idx_j.reshape(_ROWS, 128)
    partials = _sc_electro(atomic_charges, d2d, ii2d, ij2d)
    return partials[0, :_N_NODES] + partials[1, :_N_NODES]
